# baseline (speedup 1.0000x reference)
import functools

import jax
from jax import lax
import jax.numpy as jnp
from jax.experimental import pallas as pl
from jax.experimental.pallas import tpu as pltpu
from jax.experimental.pallas import tpu_sc as plsc

K = 512
D = 256
HW = 256
T = 4

NC = 2
NS = 16
NW = NC * NS
CHUNK = 128


def _screen_kernel(zb_ref, emb_ref, ids_ref, s_ref):
    zb = zb_ref[0]
    emb = emb_ref[...]

    esq = emb * emb
    e2 = (jnp.sum(esq[:, :128], axis=1) + jnp.sum(esq[:, 128:], axis=1))

    zh = zb.astype(jnp.bfloat16)
    zl = (zb - zh.astype(jnp.float32)).astype(jnp.bfloat16)
    eh = emb.astype(jnp.bfloat16)
    el = (emb - eh.astype(jnp.float32)).astype(jnp.bfloat16)

    def _dot(a, b):
        return jax.lax.dot_general(
            a, b, dimension_numbers=(((1,), (0,)), ((), ())),
            preferred_element_type=jnp.float32)

    s = _dot(eh, zh) + (_dot(eh, zl) + _dot(el, zh))
    s_ref[...] = e2.reshape(K, 1) - 2.0 * s

    def extract(t, mask_winner):
        s = s_ref[...]
        kiota = jax.lax.broadcasted_iota(jnp.int32, (K, HW), 0)
        mv = jnp.min(s, axis=0, keepdims=True)
        idx = jnp.min(jnp.where(s == mv, kiota, K),
                      axis=0, keepdims=True)
        ids_ref[0, pl.ds(t, 1), :] = idx
        if mask_winner:
            s_ref[...] = jnp.where(kiota == idx, jnp.float32(1e30), s)
        return 0

    jax.lax.fori_loop(0, T - 1, lambda t, c: extract(t, True), 0)
    extract(T - 1, False)


def _sc_gather_kernel(emb_hbm, idx_hbm, out_hbm,
                      idx_v0, idx_v1, rows_v0, rows_v1,
                      gsem0, gsem1, wsem0, wsem1):
    wid = lax.axis_index("s") * NC + lax.axis_index("c")
    n = idx_hbm.shape[0] // NW
    base = wid * n
    nchunks = n // CHUNK
    idx_bufs = [idx_v0, idx_v1]
    row_bufs = [rows_v0, rows_v1]
    gsems = [gsem0, gsem1]
    wsems = [wsem0, wsem1]
    gh = [None, None]
    wh = [None, None]
    for c in range(nchunks + 1):
        b = c & 1
        if c < nchunks:
            if wh[b] is not None:
                wh[b].wait()
            off = base + c * CHUNK
            pltpu.sync_copy(idx_hbm.at[pl.ds(off, CHUNK)], idx_bufs[b])
            gh[b] = pltpu.async_copy(emb_hbm.at[idx_bufs[b]], row_bufs[b],
                                     gsems[b])
        if c >= 1:
            pb = (c - 1) & 1
            gh[pb].wait()
            wh[pb] = pltpu.async_copy(
                row_bufs[pb], out_hbm.at[pl.ds(base + (c - 1) * CHUNK, CHUNK)],
                wsems[pb])
    wh[(nchunks - 1) & 1].wait()


def _sc_gather(emb, flat_ids):
    n_ids = flat_ids.shape[0]
    k = functools.partial(
        pl.kernel,
        mesh=plsc.VectorSubcoreMesh(core_axis_name="c", subcore_axis_name="s"),
        out_type=jax.ShapeDtypeStruct((n_ids, D), jnp.float32),
        scratch_types=[
            pltpu.VMEM((CHUNK,), jnp.int32),
            pltpu.VMEM((CHUNK,), jnp.int32),
            pltpu.VMEM((CHUNK, D), jnp.float32),
            pltpu.VMEM((CHUNK, D), jnp.float32),
            pltpu.SemaphoreType.DMA,
            pltpu.SemaphoreType.DMA,
            pltpu.SemaphoreType.DMA,
            pltpu.SemaphoreType.DMA,
        ],
    )(_sc_gather_kernel)
    return k(emb, flat_ids)


def _refine_kernel(z_ref, g_ref, ids_vec_ref, out_ref):
    z = z_ref[0]

    best_d = jnp.full((1, HW), jnp.inf, dtype=jnp.float32)
    best_i = jnp.zeros((1, HW), dtype=jnp.int32)
    for t in range(T):
        a = z - g_ref[0, t]
        sq = a * a
        d = (jnp.sum(sq[:, :128], axis=1)
             + jnp.sum(sq[:, 128:], axis=1)).reshape(1, HW)
        i = ids_vec_ref[0, t : t + 1, :]
        take = (d < best_d) | ((d == best_d) & (i < best_i))
        best_d = jnp.where(take, d, best_d)
        best_i = jnp.where(take, i, best_i)
    out_ref[0] = best_i


def _screen(zb, emb):
    b = zb.shape[0]
    hw = zb.shape[2]
    return pl.pallas_call(
        _screen_kernel,
        grid=(b,),
        in_specs=[
            pl.BlockSpec((1, D, hw), lambda i: (i, 0, 0)),
            pl.BlockSpec((K, D), lambda i: (0, 0)),
        ],
        out_specs=pl.BlockSpec((1, T, hw), lambda i: (i, 0, 0)),
        out_shape=jax.ShapeDtypeStruct((b, T, hw), jnp.int32),
        scratch_shapes=[
            pltpu.VMEM((K, hw), jnp.float32),
        ],
        compiler_params=pltpu.CompilerParams(
            dimension_semantics=("parallel",),
        ),
    )(zb, emb)


def _refine(zt, gr, ids):
    b = zt.shape[0]
    hw = zt.shape[1]
    return pl.pallas_call(
        _refine_kernel,
        grid=(b,),
        in_specs=[
            pl.BlockSpec((1, hw, D), lambda i: (i, 0, 0)),
            pl.BlockSpec((1, T, hw, D), lambda i: (i, 0, 0, 0)),
            pl.BlockSpec((1, T, hw), lambda i: (i, 0, 0)),
        ],
        out_specs=pl.BlockSpec((1, 1, hw), lambda i: (i, 0, 0)),
        out_shape=jax.ShapeDtypeStruct((b, 1, hw), jnp.int32),
        compiler_params=pltpu.CompilerParams(
            dimension_semantics=("parallel",),
        ),
    )(zt, gr, ids)


@jax.jit
def kernel(z_e_x, emb):
    B = z_e_x.shape[0]
    H, W = z_e_x.shape[2], z_e_x.shape[3]
    zb = z_e_x.reshape(B, D, H * W)

    halves = [zb[: B // 2], zb[B // 2 :]] if B % 2 == 0 else [zb]
    ids_h = [_screen(h, emb) for h in halves]
    g_h = [_sc_gather(emb, ids.reshape(-1)) for ids in ids_h]
    outs = []
    for h, ids, g in zip(halves, ids_h, g_h):
        b = h.shape[0]
        gr = g.reshape(b, T, H * W, D)
        zt = h.transpose(0, 2, 1)
        outs.append(_refine(zt, gr, ids))
    out = jnp.concatenate(outs, axis=0) if len(outs) > 1 else outs[0]
    return out.reshape(B, H, W)


# speedup vs baseline: 1.0435x; 1.0435x over previous
import functools

import jax
from jax import lax
import jax.numpy as jnp
from jax.experimental import pallas as pl
from jax.experimental.pallas import tpu as pltpu
from jax.experimental.pallas import tpu_sc as plsc

K = 512
D = 256
HW = 256
T = 4

NC = 2
NS = 16
NW = NC * NS
CHUNK = 128


def _screen_kernel(zb_ref, emb_ref, ids_ref, s_ref):
    zb = zb_ref[0]
    emb = emb_ref[...]

    esq = emb * emb
    e2 = (jnp.sum(esq[:, :128], axis=1) + jnp.sum(esq[:, 128:], axis=1))

    zh = zb.astype(jnp.bfloat16)
    zl = (zb - zh.astype(jnp.float32)).astype(jnp.bfloat16)
    eh = emb.astype(jnp.bfloat16)
    el = (emb - eh.astype(jnp.float32)).astype(jnp.bfloat16)

    def _dot(a, b):
        return jax.lax.dot_general(
            a, b, dimension_numbers=(((1,), (0,)), ((), ())),
            preferred_element_type=jnp.float32)

    s = _dot(eh, zh) + (_dot(eh, zl) + _dot(el, zh))
    s_ref[...] = e2.reshape(K, 1) - 2.0 * s

    def extract(t, mask_winner):
        s = s_ref[...]
        kiota = jax.lax.broadcasted_iota(jnp.int32, (K, HW), 0)
        mv = jnp.min(s, axis=0, keepdims=True)
        idx = jnp.min(jnp.where(s == mv, kiota, K),
                      axis=0, keepdims=True)
        ids_ref[0, pl.ds(t, 1), :] = idx
        if mask_winner:
            s_ref[...] = jnp.where(kiota == idx, jnp.float32(1e30), s)
        return 0

    jax.lax.fori_loop(0, T - 1, lambda t, c: extract(t, True), 0)
    extract(T - 1, False)


def _sc_gather_kernel(emb_hbm, idx_hbm, out_hbm, idx_v, rows_v, sem):
    wid = lax.axis_index("s") * NC + lax.axis_index("c")
    n = idx_hbm.shape[0] // NW
    base = wid * n
    for c in range(n // CHUNK):
        off = base + c * CHUNK
        pltpu.sync_copy(idx_hbm.at[pl.ds(off, CHUNK)], idx_v)
        pltpu.async_copy(emb_hbm.at[idx_v], rows_v, sem).wait()
        pltpu.sync_copy(rows_v, out_hbm.at[pl.ds(off, CHUNK)])


def _sc_gather(emb, flat_ids):
    n_ids = flat_ids.shape[0]
    k = functools.partial(
        pl.kernel,
        mesh=plsc.VectorSubcoreMesh(core_axis_name="c", subcore_axis_name="s"),
        out_type=jax.ShapeDtypeStruct((n_ids, D), jnp.float32),
        scratch_types=[
            pltpu.VMEM((CHUNK,), jnp.int32),
            pltpu.VMEM((CHUNK, D), jnp.float32),
            pltpu.SemaphoreType.DMA,
        ],
    )(_sc_gather_kernel)
    return k(emb, flat_ids)


def _refine_kernel(z_ref, g_ref, ids_vec_ref, out_ref):
    z = z_ref[0]

    best_d = jnp.full((1, HW), jnp.inf, dtype=jnp.float32)
    best_i = jnp.zeros((1, HW), dtype=jnp.int32)
    for t in range(T):
        a = z - g_ref[0, t]
        sq = a * a
        d = (jnp.sum(sq[:, :128], axis=1)
             + jnp.sum(sq[:, 128:], axis=1)).reshape(1, HW)
        i = ids_vec_ref[0, t : t + 1, :]
        take = (d < best_d) | ((d == best_d) & (i < best_i))
        best_d = jnp.where(take, d, best_d)
        best_i = jnp.where(take, i, best_i)
    out_ref[0] = best_i


def _screen(zb, emb):
    b = zb.shape[0]
    hw = zb.shape[2]
    return pl.pallas_call(
        _screen_kernel,
        grid=(b,),
        in_specs=[
            pl.BlockSpec((1, D, hw), lambda i: (i, 0, 0)),
            pl.BlockSpec((K, D), lambda i: (0, 0)),
        ],
        out_specs=pl.BlockSpec((1, T, hw), lambda i: (i, 0, 0)),
        out_shape=jax.ShapeDtypeStruct((b, T, hw), jnp.int32),
        scratch_shapes=[
            pltpu.VMEM((K, hw), jnp.float32),
        ],
        compiler_params=pltpu.CompilerParams(
            dimension_semantics=("parallel",),
        ),
    )(zb, emb)


def _refine(zt, gr, ids):
    b = zt.shape[0]
    hw = zt.shape[1]
    return pl.pallas_call(
        _refine_kernel,
        grid=(b,),
        in_specs=[
            pl.BlockSpec((1, hw, D), lambda i: (i, 0, 0)),
            pl.BlockSpec((1, T, hw, D), lambda i: (i, 0, 0, 0)),
            pl.BlockSpec((1, T, hw), lambda i: (i, 0, 0)),
        ],
        out_specs=pl.BlockSpec((1, 1, hw), lambda i: (i, 0, 0)),
        out_shape=jax.ShapeDtypeStruct((b, 1, hw), jnp.int32),
        compiler_params=pltpu.CompilerParams(
            dimension_semantics=("parallel",),
        ),
    )(zt, gr, ids)


@jax.jit
def kernel(z_e_x, emb):
    B = z_e_x.shape[0]
    H, W = z_e_x.shape[2], z_e_x.shape[3]
    zb = z_e_x.reshape(B, D, H * W)

    halves = [zb[: B // 2], zb[B // 2 :]] if B % 2 == 0 else [zb]
    ids_h = [_screen(h, emb) for h in halves]
    g_h = [_sc_gather(emb, ids.reshape(-1)) for ids in ids_h]
    outs = []
    for h, ids, g in zip(halves, ids_h, g_h):
        b = h.shape[0]
        gr = g.reshape(b, T, H * W, D)
        zt = h.transpose(0, 2, 1)
        outs.append(_refine(zt, gr, ids))
    out = jnp.concatenate(outs, axis=0) if len(outs) > 1 else outs[0]
    return out.reshape(B, H, W)
